# baseline (speedup 1.0000x reference)
import functools

import jax
import jax.numpy as jnp
from jax import lax
from jax.experimental import pallas as pl
from jax.experimental.pallas import tpu as pltpu
from jax.experimental.pallas import tpu_sc as plsc

B, N, E = 4, 1024, 8192
IN_D, ED_D = 8, 4
T, R, L, HEADS = 6, 3, 3, 8
H = 256
DH = H // HEADS
EH = H // 4
OUT = 256
RSQ = 1.0 / (DH ** 0.5)


def _rep(shape):
    nd = len(shape)
    return pl.BlockSpec(shape, lambda *args: (0,) * nd)


def _head_sel():
    r = jax.lax.broadcasted_iota(jnp.int32, (H, HEADS), 0) // DH
    c = jax.lax.broadcasted_iota(jnp.int32, (H, HEADS), 1)
    return (r == c).astype(jnp.float32)


def _typed(x, nt2, w_ref, b_ref):
    xb = x.astype(jnp.bfloat16)
    acc = jnp.zeros((N, H), jnp.float32)
    for t in range(T):
        p = jnp.dot(xb, w_ref[t].astype(jnp.bfloat16),
                    preferred_element_type=jnp.float32) + b_ref[t]
        acc = acc + jnp.where(nt2 == t, p, 0.0)
    return acc


def _ln(x, g, b):
    m = x.mean(-1, keepdims=True)
    v = ((x - m) ** 2).mean(-1, keepdims=True)
    return (x - m) / jnp.sqrt(v + 1e-5) * g + b


def _embed_body(nf_ref, ef_ref, nt_ref, tw_ref, tb_ref, ew_ref, eb_ref,
                x_ref, ee_ref):
    nf = nf_ref[0]
    nt2 = nt_ref[0]
    acc = jnp.zeros((N, H), jnp.float32)
    for t in range(T):
        p = jnp.dot(nf, tw_ref[t], preferred_element_type=jnp.float32) + tb_ref[t]
        acc = acc + jnp.where(nt2 == t, p, 0.0)
    x_ref[0] = jnp.maximum(acc, 0.0)
    ee_ref[0] = jnp.dot(ef_ref[0], ew_ref[...],
                        preferred_element_type=jnp.float32) + eb_ref[...]


def _embed(nf, ef, nt3, tw, tb, ew, eb):
    return pl.pallas_call(
        _embed_body,
        grid=(B,),
        in_specs=[
            pl.BlockSpec((1, N, IN_D), lambda b: (b, 0, 0)),
            pl.BlockSpec((1, E, ED_D), lambda b: (b, 0, 0)),
            pl.BlockSpec((1, N, 1), lambda b: (b, 0, 0)),
            _rep((T, IN_D, H)), _rep((T, H)), _rep((ED_D, EH)), _rep((1, EH)),
        ],
        out_specs=[
            pl.BlockSpec((1, N, H), lambda b: (b, 0, 0)),
            pl.BlockSpec((1, E, EH), lambda b: (b, 0, 0)),
        ],
        out_shape=[
            jax.ShapeDtypeStruct((B, N, H), jnp.float32),
            jax.ShapeDtypeStruct((B, E, EH), jnp.float32),
        ],
    )(nf, ef, nt3, tw, tb, ew, eb)


def _proj_body(x_ref, nt_ref, wk_ref, bk_ref, wq_ref, bq_ref, wv_ref, bv_ref,
               wa_ref, wm_ref, q_ref, km_ref):
    x = x_ref[0]
    nt2 = nt_ref[0]
    k = _typed(x, nt2, wk_ref, bk_ref).astype(jnp.bfloat16)
    q_ref[0] = _typed(x, nt2, wq_ref, bq_ref)
    v = _typed(x, nt2, wv_ref, bv_ref).astype(jnp.bfloat16)
    for r in range(R):
        km_ref[0, r, :, :H] = jnp.dot(
            k, wa_ref[r].astype(jnp.bfloat16),
            preferred_element_type=jnp.float32)
        km_ref[0, r, :, H:] = jnp.dot(
            v, wm_ref[r].astype(jnp.bfloat16),
            preferred_element_type=jnp.float32)


def _proj(x, nt3, wk, bk, wq, bq, wv, bv, wa, wm):
    return pl.pallas_call(
        _proj_body,
        grid=(B,),
        in_specs=[
            pl.BlockSpec((1, N, H), lambda b: (b, 0, 0)),
            pl.BlockSpec((1, N, 1), lambda b: (b, 0, 0)),
            _rep((T, H, H)), _rep((T, H)), _rep((T, H, H)), _rep((T, H)),
            _rep((T, H, H)), _rep((T, H)),
            _rep((R, H, H)), _rep((R, H, H)),
        ],
        out_specs=[
            pl.BlockSpec((1, N, H), lambda b: (b, 0, 0)),
            pl.BlockSpec((1, R, N, 2 * H), lambda b: (b, 0, 0, 0)),
        ],
        out_shape=[
            jax.ShapeDtypeStruct((B, N, H), jnp.float32),
            jax.ShapeDtypeStruct((B, R, N, 2 * H), jnp.float32),
        ],
    )(x, nt3, wk, bk, wq, bq, wv, bv, wa, wm)


SB = 2048
NSB = E // SB


def _upd_body(qg_ref, kmg_ref, ee_ref, dst_ref, x_ref, nt_ref, emw_ref,
              wo_ref, bo_ref, g_ref, b_ref, xo_ref, num_scr, den_scr):
    j = pl.program_id(1)
    sel = _head_sel()

    @pl.when(j == 0)
    def _init():
        num_scr[...] = jnp.zeros((N, H), jnp.float32)
        den_scr[...] = jnp.zeros((N, HEADS), jnp.float32)

    qg = qg_ref[0]
    kg = kmg_ref[0, :, :H]
    mg = kmg_ref[0, :, H:]
    att = jnp.dot(qg * kg, sel, preferred_element_type=jnp.float32) * RSQ
    p = jnp.exp(att)
    pe = jnp.dot(p, sel.T, preferred_element_type=jnp.float32)
    em = jnp.dot(ee_ref[0], emw_ref[...], preferred_element_type=jnp.float32)
    y = (mg + em) * pe
    ion = jax.lax.broadcasted_iota(jnp.int32, (N, 1), 0)
    mask = (ion == dst_ref[0])
    maskf = mask.astype(jnp.float32)
    num_scr[...] += jnp.dot(mask.astype(jnp.bfloat16),
                            y.astype(jnp.bfloat16),
                            preferred_element_type=jnp.float32)
    den_scr[...] += jnp.dot(maskf, p, preferred_element_type=jnp.float32)

    @pl.when(j == NSB - 1)
    def _fin():
        den_b = jnp.dot(den_scr[...], sel.T,
                        preferred_element_type=jnp.float32)
        agg = num_scr[...] / (den_b + 1e-9)
        ga = jax.nn.gelu(agg)
        out = _typed(ga, nt_ref[0], wo_ref, bo_ref)
        xo_ref[0] = _ln(out + x_ref[0], g_ref[...], b_ref[...])


def _upd(qg, kmg, ee, dst3, x, nt3, emw, wo, bo, g, b):
    return pl.pallas_call(
        _upd_body,
        grid=(B, NSB),
        in_specs=[
            pl.BlockSpec((1, SB, H), lambda b, j: (b, j, 0)),
            pl.BlockSpec((1, SB, 2 * H), lambda b, j: (b, j, 0)),
            pl.BlockSpec((1, SB, EH), lambda b, j: (b, j, 0)),
            pl.BlockSpec((1, 1, SB), lambda b, j: (b, 0, j)),
            pl.BlockSpec((1, N, H), lambda b, j: (b, 0, 0)),
            pl.BlockSpec((1, N, 1), lambda b, j: (b, 0, 0)),
            _rep((EH, H)),
            _rep((T, H, H)), _rep((T, H)), _rep((1, H)), _rep((1, H)),
        ],
        out_specs=pl.BlockSpec((1, N, H), lambda b, j: (b, 0, 0)),
        out_shape=jax.ShapeDtypeStruct((B, N, H), jnp.float32),
        scratch_shapes=[pltpu.VMEM((N, H), jnp.float32),
                        pltpu.VMEM((N, HEADS), jnp.float32)],
    )(qg, kmg, ee, dst3, x, nt3, emw, wo, bo, g, b)


NW = 32
ET = B * E
EPW = ET // NW
CH = 128
NCH = EPW // CH
BN = B * N
RPS = BN // 16

_SC_MESH = plsc.VectorSubcoreMesh(core_axis_name="c", subcore_axis_name="s")


def _gather_body(q_hbm, km_hbm, src_hbm, dst_hbm, et_hbm, qg_hbm, kmg_hbm,
                 srcv, dstv, etv, qi, kmi, qrows, kmrows, sem):
    c = lax.axis_index("c")
    s = lax.axis_index("s")
    wid = s * 2 + c
    base = pl.multiple_of(wid * EPW, 1024)
    b = wid // (E // EPW)
    pltpu.sync_copy(src_hbm.at[pl.ds(base, EPW)], srcv)
    pltpu.sync_copy(dst_hbm.at[pl.ds(base, EPW)], dstv)
    pltpu.sync_copy(et_hbm.at[pl.ds(base, EPW)], etv)
    for j in range(NCH):
        for i in range(CH // 16):
            o = j * CH + i * 16
            sv = srcv[pl.ds(o, 16)]
            dv = dstv[pl.ds(o, 16)]
            ev = etv[pl.ds(o, 16)]
            qi[j, pl.ds(i * 16, 16)] = dv + b * N
            kmi[j, pl.ds(i * 16, 16)] = (b * R + ev) * N + sv
    for j in range(NCH):
        pltpu.async_copy(q_hbm.at[qi.at[j]], qrows, sem).wait()
        pltpu.async_copy(km_hbm.at[kmi.at[j]], kmrows, sem).wait()
        pltpu.sync_copy(qrows, qg_hbm.at[pl.ds(base + j * CH, CH)])
        pltpu.sync_copy(kmrows, kmg_hbm.at[pl.ds(base + j * CH, CH)])


_gather_sc = functools.partial(
    pl.kernel, _gather_body, mesh=_SC_MESH,
    out_type=[
        jax.ShapeDtypeStruct((ET, H), jnp.float32),
        jax.ShapeDtypeStruct((ET, 2 * H), jnp.float32),
    ],
    scratch_types=[
        pltpu.VMEM((EPW,), jnp.int32),
        pltpu.VMEM((EPW,), jnp.int32),
        pltpu.VMEM((EPW,), jnp.int32),
        pltpu.VMEM((NCH, CH), jnp.int32),
        pltpu.VMEM((NCH, CH), jnp.int32),
        pltpu.VMEM((CH, H), jnp.float32),
        pltpu.VMEM((CH, 2 * H), jnp.float32),
        pltpu.SemaphoreType.DMA,
    ],
)()


def _haz_body(x_ref, nt_ref, wq_ref, wk_ref, wv_ref, wo_ref, hb_ref,
              w1_ref, b1_ref, w2_ref, b2_ref, out_ref, o_scr):
    x = x_ref[0]
    nt2 = nt_ref[0]
    qh = jnp.dot(x, wq_ref[...], preferred_element_type=jnp.float32)
    kh = jnp.dot(x, wk_ref[...], preferred_element_type=jnp.float32)
    vh = jnp.dot(x, wv_ref[...], preferred_element_type=jnp.float32)
    tio = jax.lax.broadcasted_iota(jnp.int32, (N, T), 1)
    oh = (nt2 == tio).astype(jnp.float32)
    bq = jnp.dot(oh, hb_ref[...], preferred_element_type=jnp.float32)
    bias = jax.lax.dot_general(bq, oh, (((1,), (1,)), ((), ())),
                               preferred_element_type=jnp.float32)
    for h in range(HEADS):
        qs = qh[:, h * DH:(h + 1) * DH]
        ks = kh[:, h * DH:(h + 1) * DH]
        vs = vh[:, h * DH:(h + 1) * DH]
        lg = jax.lax.dot_general(qs, ks, (((1,), (1,)), ((), ())),
                                 preferred_element_type=jnp.float32) * RSQ
        lg = lg + bias
        lg = lg - jnp.max(lg, axis=-1, keepdims=True)
        el = jnp.exp(lg)
        a = el / jnp.sum(el, axis=-1, keepdims=True)
        o_scr[:, h * DH:(h + 1) * DH] = jnp.dot(
            a, vs, preferred_element_type=jnp.float32)
    y = jnp.dot(o_scr[...], wo_ref[...], preferred_element_type=jnp.float32)
    mean = jnp.sum(y, axis=0, keepdims=True) / N
    mx = jnp.max(y, axis=0, keepdims=True)
    pooled = jnp.concatenate([mean, mx], axis=1)
    h1 = jnp.maximum(jnp.dot(pooled, w1_ref[...],
                             preferred_element_type=jnp.float32)
                     + b1_ref[...], 0.0)
    out_ref[0] = jnp.dot(h1, w2_ref[...],
                         preferred_element_type=jnp.float32) + b2_ref[...]


def _hazard(x, nt3, wq, wk, wv, wo, hb, w1, b1, w2, b2):
    return pl.pallas_call(
        _haz_body,
        grid=(B,),
        in_specs=[
            pl.BlockSpec((1, N, H), lambda b: (b, 0, 0)),
            pl.BlockSpec((1, N, 1), lambda b: (b, 0, 0)),
            _rep((H, H)), _rep((H, H)), _rep((H, H)), _rep((H, H)),
            _rep((T, T)),
            _rep((2 * H, H)), _rep((1, H)), _rep((H, OUT)), _rep((1, OUT)),
        ],
        out_specs=pl.BlockSpec((1, 1, OUT), lambda b: (b, 0, 0)),
        out_shape=jax.ShapeDtypeStruct((B, 1, OUT), jnp.float32),
        scratch_shapes=[pltpu.VMEM((N, H), jnp.float32)],
    )(x, nt3, wq, wk, wv, wo, hb, w1, b1, w2, b2)


def kernel(graph_node_feats, graph_edge_feats, graph_edge_index,
           graph_node_types, graph_edge_types, graph_node_mask,
           graph_edge_mask, type_emb_W, type_emb_b, edge_emb_W, edge_emb_b,
           Wk, bk, Wq, bq, Wv, bv, W_att, W_msg, mu, W_out, b_out,
           edge_msg_W, ln_g, ln_b, haz_Wq, haz_Wk, haz_Wv, haz_Wo, haz_bias,
           out_W1, out_b1, out_W2, out_b2):
    nt3 = graph_node_types.astype(jnp.int32).reshape(B, N, 1)
    src = graph_edge_index[:, 0, :].astype(jnp.int32)
    dst = graph_edge_index[:, 1, :].astype(jnp.int32)
    et = graph_edge_types.astype(jnp.int32)

    hi = jnp.arange(H) // DH
    blk = (hi[:, None] == hi[None, :]).astype(jnp.float32)
    wa_full = jnp.einsum('lrhdf->lrhdf', W_att)
    def blockdiag(w):
        z = jnp.zeros((L, R, H, H), jnp.float32)
        for h in range(HEADS):
            z = z.at[:, :, h * DH:(h + 1) * DH, h * DH:(h + 1) * DH].set(
                w[:, :, h])
        return z
    wa = blockdiag(W_att * mu[:, :, :, None, None])
    wm = blockdiag(W_msg)

    x, ee = _embed(graph_node_feats, graph_edge_feats, nt3,
                   type_emb_W, type_emb_b, edge_emb_W,
                   edge_emb_b.reshape(1, EH))

    srcf = src.reshape(ET)
    dstf = dst.reshape(ET)
    etf = et.reshape(ET)
    dst3 = dst.reshape(B, 1, E)
    for l in range(L):
        q, km = _proj(x, nt3, Wk[l], bk[l], Wq[l], bq[l], Wv[l], bv[l],
                      wa[l], wm[l])
        qg, kmg = _gather_sc(q.reshape(BN, H), km.reshape(B * R * N, 2 * H),
                             srcf, dstf, etf)
        x = _upd(qg.reshape(B, E, H), kmg.reshape(B, E, 2 * H), ee, dst3,
                 x, nt3, edge_msg_W[l], W_out[l], b_out[l],
                 ln_g[l].reshape(1, H), ln_b[l].reshape(1, H))

    return _hazard(x, nt3, haz_Wq, haz_Wk, haz_Wv, haz_Wo, haz_bias,
                   out_W1, out_b1.reshape(1, H), out_W2,
                   out_b2.reshape(1, OUT)).reshape(B, OUT)


# speedup vs baseline: 1.1606x; 1.1606x over previous
import functools

import jax
import jax.numpy as jnp
from jax import lax
from jax.experimental import pallas as pl
from jax.experimental.pallas import tpu as pltpu
from jax.experimental.pallas import tpu_sc as plsc

B, N, E = 4, 1024, 8192
IN_D, ED_D = 8, 4
T, R, L, HEADS = 6, 3, 3, 8
H = 256
DH = H // HEADS
EH = H // 4
OUT = 256
RSQ = 1.0 / (DH ** 0.5)


def _rep(shape):
    nd = len(shape)
    return pl.BlockSpec(shape, lambda *args: (0,) * nd)


def _head_sel():
    r = jax.lax.broadcasted_iota(jnp.int32, (H, HEADS), 0) // DH
    c = jax.lax.broadcasted_iota(jnp.int32, (H, HEADS), 1)
    return (r == c).astype(jnp.float32)


def _typed(x, nt2, w_ref, b_ref):
    xb = x.astype(jnp.bfloat16)
    acc = jnp.zeros((N, H), jnp.float32)
    for t in range(T):
        p = jnp.dot(xb, w_ref[t].astype(jnp.bfloat16),
                    preferred_element_type=jnp.float32) + b_ref[t]
        acc = acc + jnp.where(nt2 == t, p, 0.0)
    return acc


def _pack_pair(a, b):
    ua = jax.lax.bitcast_convert_type(a, jnp.uint32)
    ub = jax.lax.bitcast_convert_type(b, jnp.uint32)
    ra = ua + 0x7FFF + ((ua >> 16) & 1)
    rb = ub + 0x7FFF + ((ub >> 16) & 1)
    w = (ra & jnp.uint32(0xFFFF0000)) | (rb >> 16)
    return jax.lax.bitcast_convert_type(w, jnp.float32)


def _unpack_pair(w):
    u = jax.lax.bitcast_convert_type(w, jnp.uint32)
    hi = jax.lax.bitcast_convert_type(u & jnp.uint32(0xFFFF0000), jnp.float32)
    lo = jax.lax.bitcast_convert_type(u << 16, jnp.float32)
    return jnp.concatenate([hi, lo], axis=1)


def _ln(x, g, b):
    m = x.mean(-1, keepdims=True)
    v = ((x - m) ** 2).mean(-1, keepdims=True)
    return (x - m) / jnp.sqrt(v + 1e-5) * g + b


def _embed_body(nf_ref, ef_ref, nt_ref, tw_ref, tb_ref, ew_ref, eb_ref,
                x_ref, ee_ref):
    nf = nf_ref[0]
    nt2 = nt_ref[0]
    acc = jnp.zeros((N, H), jnp.float32)
    for t in range(T):
        p = jnp.dot(nf, tw_ref[t], preferred_element_type=jnp.float32) + tb_ref[t]
        acc = acc + jnp.where(nt2 == t, p, 0.0)
    x_ref[0] = jnp.maximum(acc, 0.0)
    ee_ref[0] = jnp.dot(ef_ref[0], ew_ref[...],
                        preferred_element_type=jnp.float32) + eb_ref[...]


def _embed(nf, ef, nt3, tw, tb, ew, eb):
    return pl.pallas_call(
        _embed_body,
        grid=(B,),
        in_specs=[
            pl.BlockSpec((1, N, IN_D), lambda b: (b, 0, 0)),
            pl.BlockSpec((1, E, ED_D), lambda b: (b, 0, 0)),
            pl.BlockSpec((1, N, 1), lambda b: (b, 0, 0)),
            _rep((T, IN_D, H)), _rep((T, H)), _rep((ED_D, EH)), _rep((1, EH)),
        ],
        out_specs=[
            pl.BlockSpec((1, N, H), lambda b: (b, 0, 0)),
            pl.BlockSpec((1, E, EH), lambda b: (b, 0, 0)),
        ],
        out_shape=[
            jax.ShapeDtypeStruct((B, N, H), jnp.float32),
            jax.ShapeDtypeStruct((B, E, EH), jnp.float32),
        ],
    )(nf, ef, nt3, tw, tb, ew, eb)


def _proj_body(x_ref, nt_ref, wk_ref, bk_ref, wq_ref, bq_ref, wv_ref, bv_ref,
               wa_ref, wm_ref, q_ref, km_ref):
    x = x_ref[0]
    nt2 = nt_ref[0]
    k = _typed(x, nt2, wk_ref, bk_ref).astype(jnp.bfloat16)
    q = _typed(x, nt2, wq_ref, bq_ref)
    q_ref[0] = _pack_pair(q[:, :H // 2], q[:, H // 2:])
    v = _typed(x, nt2, wv_ref, bv_ref).astype(jnp.bfloat16)
    for r in range(R):
        kt = jnp.dot(k, wa_ref[r].astype(jnp.bfloat16),
                     preferred_element_type=jnp.float32)
        mt = jnp.dot(v, wm_ref[r].astype(jnp.bfloat16),
                     preferred_element_type=jnp.float32)
        km_ref[0, r, :, :H // 2] = _pack_pair(kt[:, :H // 2], kt[:, H // 2:])
        km_ref[0, r, :, H // 2:] = _pack_pair(mt[:, :H // 2], mt[:, H // 2:])


def _proj(x, nt3, wk, bk, wq, bq, wv, bv, wa, wm):
    return pl.pallas_call(
        _proj_body,
        grid=(B,),
        in_specs=[
            pl.BlockSpec((1, N, H), lambda b: (b, 0, 0)),
            pl.BlockSpec((1, N, 1), lambda b: (b, 0, 0)),
            _rep((T, H, H)), _rep((T, H)), _rep((T, H, H)), _rep((T, H)),
            _rep((T, H, H)), _rep((T, H)),
            _rep((R, H, H)), _rep((R, H, H)),
        ],
        out_specs=[
            pl.BlockSpec((1, N, H // 2), lambda b: (b, 0, 0)),
            pl.BlockSpec((1, R, N, H), lambda b: (b, 0, 0, 0)),
        ],
        out_shape=[
            jax.ShapeDtypeStruct((B, N, H // 2), jnp.float32),
            jax.ShapeDtypeStruct((B, R, N, H), jnp.float32),
        ],
    )(x, nt3, wk, bk, wq, bq, wv, bv, wa, wm)


SB = 2048
NSB = E // SB


def _upd_body(qg_ref, kmg_ref, ee_ref, dst_ref, x_ref, nt_ref, emw_ref,
              wo_ref, bo_ref, g_ref, b_ref, xo_ref, num_scr, den_scr):
    j = pl.program_id(1)
    sel = _head_sel()

    @pl.when(j == 0)
    def _init():
        num_scr[...] = jnp.zeros((N, H), jnp.float32)
        den_scr[...] = jnp.zeros((N, HEADS), jnp.float32)

    qg = _unpack_pair(qg_ref[0])
    kg = _unpack_pair(kmg_ref[0, :, :H // 2])
    mg = _unpack_pair(kmg_ref[0, :, H // 2:])
    att = jnp.dot(qg * kg, sel, preferred_element_type=jnp.float32) * RSQ
    p = jnp.exp(att)
    pe = jnp.dot(p, sel.T, preferred_element_type=jnp.float32)
    em = jnp.dot(ee_ref[0], emw_ref[...], preferred_element_type=jnp.float32)
    y = (mg + em) * pe
    ion = jax.lax.broadcasted_iota(jnp.int32, (N, 1), 0)
    mask = (ion == dst_ref[0])
    maskf = mask.astype(jnp.float32)
    num_scr[...] += jnp.dot(mask.astype(jnp.bfloat16),
                            y.astype(jnp.bfloat16),
                            preferred_element_type=jnp.float32)
    den_scr[...] += jnp.dot(maskf, p, preferred_element_type=jnp.float32)

    @pl.when(j == NSB - 1)
    def _fin():
        den_b = jnp.dot(den_scr[...], sel.T,
                        preferred_element_type=jnp.float32)
        agg = num_scr[...] / (den_b + 1e-9)
        ga = jax.nn.gelu(agg)
        out = _typed(ga, nt_ref[0], wo_ref, bo_ref)
        xo_ref[0] = _ln(out + x_ref[0], g_ref[...], b_ref[...])


def _upd(qg, kmg, ee, dst3, x, nt3, emw, wo, bo, g, b):
    return pl.pallas_call(
        _upd_body,
        grid=(B, NSB),
        in_specs=[
            pl.BlockSpec((1, SB, H // 2), lambda b, j: (b, j, 0)),
            pl.BlockSpec((1, SB, H), lambda b, j: (b, j, 0)),
            pl.BlockSpec((1, SB, EH), lambda b, j: (b, j, 0)),
            pl.BlockSpec((1, 1, SB), lambda b, j: (b, 0, j)),
            pl.BlockSpec((1, N, H), lambda b, j: (b, 0, 0)),
            pl.BlockSpec((1, N, 1), lambda b, j: (b, 0, 0)),
            _rep((EH, H)),
            _rep((T, H, H)), _rep((T, H)), _rep((1, H)), _rep((1, H)),
        ],
        out_specs=pl.BlockSpec((1, N, H), lambda b, j: (b, 0, 0)),
        out_shape=jax.ShapeDtypeStruct((B, N, H), jnp.float32),
        scratch_shapes=[pltpu.VMEM((N, H), jnp.float32),
                        pltpu.VMEM((N, HEADS), jnp.float32)],
    )(qg, kmg, ee, dst3, x, nt3, emw, wo, bo, g, b)


NW = 32
ET = B * E
EPW = ET // NW
CH = 128
NCH = EPW // CH
BN = B * N
RPS = BN // 16

_SC_MESH = plsc.VectorSubcoreMesh(core_axis_name="c", subcore_axis_name="s")


def _gather_body(q_hbm, km_hbm, src_hbm, dst_hbm, et_hbm, qg_hbm, kmg_hbm,
                 srcv, dstv, etv, qi, kmi, qrows, kmrows, sem):
    c = lax.axis_index("c")
    s = lax.axis_index("s")
    wid = s * 2 + c
    base = pl.multiple_of(wid * EPW, 1024)
    b = wid // (E // EPW)
    pltpu.sync_copy(src_hbm.at[pl.ds(base, EPW)], srcv)
    pltpu.sync_copy(dst_hbm.at[pl.ds(base, EPW)], dstv)
    pltpu.sync_copy(et_hbm.at[pl.ds(base, EPW)], etv)
    for j in range(NCH):
        for i in range(CH // 16):
            o = j * CH + i * 16
            sv = srcv[pl.ds(o, 16)]
            dv = dstv[pl.ds(o, 16)]
            ev = etv[pl.ds(o, 16)]
            qi[j, pl.ds(i * 16, 16)] = dv + b * N
            kmi[j, pl.ds(i * 16, 16)] = (b * R + ev) * N + sv
    for j in range(NCH):
        pltpu.async_copy(q_hbm.at[qi.at[j]], qrows, sem).wait()
        pltpu.async_copy(km_hbm.at[kmi.at[j]], kmrows, sem).wait()
        pltpu.sync_copy(qrows, qg_hbm.at[pl.ds(base + j * CH, CH)])
        pltpu.sync_copy(kmrows, kmg_hbm.at[pl.ds(base + j * CH, CH)])


_gather_sc = functools.partial(
    pl.kernel, _gather_body, mesh=_SC_MESH,
    out_type=[
        jax.ShapeDtypeStruct((ET, H // 2), jnp.float32),
        jax.ShapeDtypeStruct((ET, H), jnp.float32),
    ],
    scratch_types=[
        pltpu.VMEM((EPW,), jnp.int32),
        pltpu.VMEM((EPW,), jnp.int32),
        pltpu.VMEM((EPW,), jnp.int32),
        pltpu.VMEM((NCH, CH), jnp.int32),
        pltpu.VMEM((NCH, CH), jnp.int32),
        pltpu.VMEM((CH, H // 2), jnp.float32),
        pltpu.VMEM((CH, H), jnp.float32),
        pltpu.SemaphoreType.DMA,
    ],
)()


def _haz_body(x_ref, nt_ref, wq_ref, wk_ref, wv_ref, wo_ref, hb_ref,
              w1_ref, b1_ref, w2_ref, b2_ref, out_ref, o_scr):
    x = x_ref[0]
    nt2 = nt_ref[0]
    qh = jnp.dot(x, wq_ref[...], preferred_element_type=jnp.float32)
    kh = jnp.dot(x, wk_ref[...], preferred_element_type=jnp.float32)
    vh = jnp.dot(x, wv_ref[...], preferred_element_type=jnp.float32)
    tio = jax.lax.broadcasted_iota(jnp.int32, (N, T), 1)
    oh = (nt2 == tio).astype(jnp.float32)
    bq = jnp.dot(oh, hb_ref[...], preferred_element_type=jnp.float32)
    bias = jax.lax.dot_general(bq, oh, (((1,), (1,)), ((), ())),
                               preferred_element_type=jnp.float32)
    for h in range(HEADS):
        qs = qh[:, h * DH:(h + 1) * DH]
        ks = kh[:, h * DH:(h + 1) * DH]
        vs = vh[:, h * DH:(h + 1) * DH]
        lg = jax.lax.dot_general(qs, ks, (((1,), (1,)), ((), ())),
                                 preferred_element_type=jnp.float32) * RSQ
        lg = lg + bias
        lg = lg - jnp.max(lg, axis=-1, keepdims=True)
        el = jnp.exp(lg)
        a = el / jnp.sum(el, axis=-1, keepdims=True)
        o_scr[:, h * DH:(h + 1) * DH] = jnp.dot(
            a, vs, preferred_element_type=jnp.float32)
    y = jnp.dot(o_scr[...], wo_ref[...], preferred_element_type=jnp.float32)
    mean = jnp.sum(y, axis=0, keepdims=True) / N
    mx = jnp.max(y, axis=0, keepdims=True)
    pooled = jnp.concatenate([mean, mx], axis=1)
    h1 = jnp.maximum(jnp.dot(pooled, w1_ref[...],
                             preferred_element_type=jnp.float32)
                     + b1_ref[...], 0.0)
    out_ref[0] = jnp.dot(h1, w2_ref[...],
                         preferred_element_type=jnp.float32) + b2_ref[...]


def _hazard(x, nt3, wq, wk, wv, wo, hb, w1, b1, w2, b2):
    return pl.pallas_call(
        _haz_body,
        grid=(B,),
        in_specs=[
            pl.BlockSpec((1, N, H), lambda b: (b, 0, 0)),
            pl.BlockSpec((1, N, 1), lambda b: (b, 0, 0)),
            _rep((H, H)), _rep((H, H)), _rep((H, H)), _rep((H, H)),
            _rep((T, T)),
            _rep((2 * H, H)), _rep((1, H)), _rep((H, OUT)), _rep((1, OUT)),
        ],
        out_specs=pl.BlockSpec((1, 1, OUT), lambda b: (b, 0, 0)),
        out_shape=jax.ShapeDtypeStruct((B, 1, OUT), jnp.float32),
        scratch_shapes=[pltpu.VMEM((N, H), jnp.float32)],
    )(x, nt3, wq, wk, wv, wo, hb, w1, b1, w2, b2)


def kernel(graph_node_feats, graph_edge_feats, graph_edge_index,
           graph_node_types, graph_edge_types, graph_node_mask,
           graph_edge_mask, type_emb_W, type_emb_b, edge_emb_W, edge_emb_b,
           Wk, bk, Wq, bq, Wv, bv, W_att, W_msg, mu, W_out, b_out,
           edge_msg_W, ln_g, ln_b, haz_Wq, haz_Wk, haz_Wv, haz_Wo, haz_bias,
           out_W1, out_b1, out_W2, out_b2):
    nt3 = graph_node_types.astype(jnp.int32).reshape(B, N, 1)
    src = graph_edge_index[:, 0, :].astype(jnp.int32)
    dst = graph_edge_index[:, 1, :].astype(jnp.int32)
    et = graph_edge_types.astype(jnp.int32)

    hi = jnp.arange(H) // DH
    blk = (hi[:, None] == hi[None, :]).astype(jnp.float32)
    wa_full = jnp.einsum('lrhdf->lrhdf', W_att)
    def blockdiag(w):
        z = jnp.zeros((L, R, H, H), jnp.float32)
        for h in range(HEADS):
            z = z.at[:, :, h * DH:(h + 1) * DH, h * DH:(h + 1) * DH].set(
                w[:, :, h])
        return z
    wa = blockdiag(W_att * mu[:, :, :, None, None])
    wm = blockdiag(W_msg)

    x, ee = _embed(graph_node_feats, graph_edge_feats, nt3,
                   type_emb_W, type_emb_b, edge_emb_W,
                   edge_emb_b.reshape(1, EH))

    srcf = src.reshape(ET)
    dstf = dst.reshape(ET)
    etf = et.reshape(ET)
    dst3 = dst.reshape(B, 1, E)
    for l in range(L):
        q, km = _proj(x, nt3, Wk[l], bk[l], Wq[l], bq[l], Wv[l], bv[l],
                      wa[l], wm[l])
        qg, kmg = _gather_sc(q.reshape(BN, H // 2),
                             km.reshape(B * R * N, H), srcf, dstf, etf)
        x = _upd(qg.reshape(B, E, H // 2), kmg.reshape(B, E, H), ee, dst3,
                 x, nt3, edge_msg_W[l], W_out[l], b_out[l],
                 ln_g[l].reshape(1, H), ln_b[l].reshape(1, H))

    return _hazard(x, nt3, haz_Wq, haz_Wk, haz_Wv, haz_Wo, haz_bias,
                   out_W1, out_b1.reshape(1, H), out_W2,
                   out_b2.reshape(1, OUT)).reshape(B, OUT)
